# baseline (speedup 1.0000x reference)
import jax
import jax.numpy as jnp
from jax import lax
from jax.experimental import pallas as pl
from jax.experimental.pallas import tpu as pltpu
from jax.experimental.pallas import tpu_sc as plsc

BOHR = 0.52917721067121
ZMAX = 92
NC = 2
NS = 16
NW = NC * NS
LANES = 16
BLK = 128
NBLK = 16
CHUNK = NBLK * BLK
UNROLL = 16


def _pad1d(x, mult):
    n = x.shape[0]
    p = (-n) % mult
    if p:
        x = jnp.concatenate([x, jnp.zeros((p,), x.dtype)])
    return x


def _make_sc_call(n_nodes, n_edges, spk_n, tab_n):
    nch = n_edges // CHUNK
    base, rem = nch // NW, nch % NW

    def body(spk_h, csf_h, alf_h, src_h, dst_h, d_h, sw_h, zeros_h,
             out_h,
             spk, csf, alf,
             srcb0, srcb1, dstb0, dstb1, db0, db1, swb0, swb1,
             valb0, valb1, scb0, scb1,
             acc, sem0, sem1, scsem0, scsem1):
        cid = lax.axis_index("c")
        sid = lax.axis_index("s")
        wid = sid * NC + cid
        start = wid * base + jnp.minimum(wid, rem)
        cnt = base + (wid < rem).astype(jnp.int32)

        pltpu.sync_copy(spk_h, spk)
        pltpu.sync_copy(csf_h, csf)
        pltpu.sync_copy(alf_h, alf)

        @pl.when(sid == 0)
        def _():
            pltpu.sync_copy(zeros_h, acc)
        plsc.subcore_barrier()

        ins = (src_h, dst_h, d_h, sw_h)
        bufs0 = (srcb0, dstb0, db0, swb0)
        bufs1 = (srcb1, dstb1, db1, swb1)

        def fire_in(c, bufs, sem):
            row = (start + c) * NBLK
            for h, b in zip(ins, bufs):
                pltpu.async_copy(h.at[pl.ds(row, NBLK)], b, sem)

        def wait_in(bufs, sem):
            for h, b in zip(ins, bufs):
                pltpu.make_async_copy(h.at[pl.ds(0, NBLK)], b, sem).wait()

        def fire_sc(valb, scb, sem):
            for j in range(NBLK):
                pltpu.async_copy(valb.at[pl.ds(j * BLK, BLK)],
                                 acc.at[scb.at[j]], sem, add=True)

        def wait_sc(valb, scb, sem):
            for j in range(NBLK):
                pltpu.make_async_copy(valb.at[pl.ds(j * BLK, BLK)],
                                      acc.at[scb.at[j]], sem).wait()

        half = 0.5 * BOHR

        def compute(srcb, dstb, db, swb, valb, scb):
            @plsc.parallel_loop(0, CHUNK // LANES, 1, unroll=UNROLL)
            def _(g):
                j = lax.shift_right_logical(g, 3)
                sl = pl.ds((g & 7) * LANES, LANES)
                s = srcb[j, sl]
                t = dstb[j, sl]
                zs_p = plsc.load_gather(spk, [lax.shift_right_logical(s, 2)])
                zt_p = plsc.load_gather(spk, [lax.shift_right_logical(t, 2)])
                zs = lax.shift_right_logical(zs_p, (s & 3) * 8) & 255
                zt = lax.shift_right_logical(zt_p, (t & 3) * 8) & 255
                b3 = (zs + ZMAX * zt) * 3
                d = db[j, sl]
                nd = -d
                phi = (plsc.load_gather(csf, [b3])
                       * jnp.exp(plsc.load_gather(alf, [b3]) * nd)
                       + plsc.load_gather(csf, [b3 + 1])
                       * jnp.exp(plsc.load_gather(alf, [b3 + 1]) * nd)
                       + plsc.load_gather(csf, [b3 + 2])
                       * jnp.exp(plsc.load_gather(alf, [b3 + 2]) * nd))
                zz = (zs * zt).astype(jnp.float32)
                scb[j, sl] = s
                valb[pl.ds(g * LANES, LANES)] = zz * phi * swb[j, sl] * half / d

        @pl.when(cnt > 0)
        def _():
            fire_in(0, bufs0, sem0)

        npairs = (cnt + 1) // 2

        def pair(p, carry):
            c1 = 2 * p + 1
            c2 = 2 * p + 2

            @pl.when(c1 < cnt)
            def _():
                fire_in(c1, bufs1, sem1)

            wait_in(bufs0, sem0)

            @pl.when(p > 0)
            def _():
                wait_sc(valb0, scb0, scsem0)
            compute(srcb0, dstb0, db0, swb0, valb0, scb0)
            fire_sc(valb0, scb0, scsem0)

            @pl.when(c2 < cnt)
            def _():
                fire_in(c2, bufs0, sem0)

            @pl.when(c1 < cnt)
            def _():
                wait_in(bufs1, sem1)

                @pl.when(p > 0)
                def _():
                    wait_sc(valb1, scb1, scsem1)
                compute(srcb1, dstb1, db1, swb1, valb1, scb1)
                fire_sc(valb1, scb1, scsem1)
            return carry

        lax.fori_loop(0, npairs, pair, 0)

        @pl.when(cnt >= 1)
        def _():
            wait_sc(valb0, scb0, scsem0)

        @pl.when(cnt >= 2)
        def _():
            wait_sc(valb1, scb1, scsem1)

        plsc.subcore_barrier()

        @pl.when(sid == 0)
        def _():
            pltpu.sync_copy(acc, out_h.at[cid])

    f32 = jnp.float32
    i32 = jnp.int32
    mesh = plsc.VectorSubcoreMesh(core_axis_name="c", subcore_axis_name="s",
                                  num_cores=NC, num_subcores=NS)
    ebuf_i = pltpu.VMEM((NBLK, BLK), i32)
    ebuf_f = pltpu.VMEM((NBLK, BLK), f32)
    vbuf_f = pltpu.VMEM((CHUNK,), f32)
    return pl.kernel(
        body,
        out_type=jax.ShapeDtypeStruct((NC, n_nodes), f32),
        mesh=mesh,
        compiler_params=pltpu.CompilerParams(needs_layout_passes=False),
        scratch_types=[
            pltpu.VMEM((spk_n,), i32),
            pltpu.VMEM((tab_n,), f32),
            pltpu.VMEM((tab_n,), f32),
            ebuf_i, ebuf_i, ebuf_i, ebuf_i, ebuf_f, ebuf_f, ebuf_f, ebuf_f,
            vbuf_f, vbuf_f, ebuf_i, ebuf_i,
            pltpu.VMEM_SHARED((n_nodes,), f32),
            pltpu.SemaphoreType.DMA,
            pltpu.SemaphoreType.DMA,
            pltpu.SemaphoreType.DMA,
            pltpu.SemaphoreType.DMA,
        ],
    )


def kernel(species, edge_src, edge_dst, distances, switch, CS, ALPHAS):
    n_nodes = species.shape[0]
    n_edges = edge_src.shape[0]
    i32 = jnp.int32

    sp = _pad1d(species.astype(i32), 4).reshape(-1, 4)
    packed = (sp[:, 0] | (sp[:, 1] << 8) | (sp[:, 2] << 16) | (sp[:, 3] << 24))
    packed = _pad1d(packed, 16)

    csf = _pad1d(CS.reshape(-1), 32)
    alf = _pad1d(ALPHAS.reshape(-1), 32)

    src2 = edge_src.astype(i32).reshape(-1, BLK)
    dst2 = edge_dst.astype(i32).reshape(-1, BLK)
    d2 = distances.reshape(-1, BLK)
    sw2 = switch.reshape(-1, BLK)
    zeros = jnp.zeros((n_nodes,), jnp.float32)

    call = _make_sc_call(n_nodes, n_edges, packed.shape[0], csf.shape[0])
    out = call(packed, csf, alf, src2, dst2, d2, sw2, zeros)
    return out[0] + out[1]


# speedup vs baseline: 1.8303x; 1.8303x over previous
import jax
import jax.numpy as jnp
from jax import lax
from jax.experimental import pallas as pl
from jax.experimental.pallas import tpu as pltpu
from jax.experimental.pallas import tpu_sc as plsc

BOHR = 0.52917721067121
ZMAX = 92
NC = 2
NS = 16
NW = NC * NS
LANES = 16
BLK = 128
NBLK = 16
CHUNK = NBLK * BLK
UNROLL = 8


def _pad1d(x, mult):
    n = x.shape[0]
    p = (-n) % mult
    if p:
        x = jnp.concatenate([x, jnp.zeros((p,), x.dtype)])
    return x


def _make_sc_call(n_nodes, n_edges, spk_n, tab_n):
    nch = n_edges // CHUNK
    base, rem = nch // NW, nch % NW

    def body(spk_h, pk_h, src_h, dst_h, d_h, sw_h, zeros_h,
             out_h,
             spk, pk,
             srcb0, srcb1, dstb0, dstb1, db0, db1, swb0, swb1,
             valb0, valb1, scb0, scb1,
             acc, sem0, sem1, scsem0, scsem1):
        cid = lax.axis_index("c")
        sid = lax.axis_index("s")
        wid = sid * NC + cid
        start = wid * base + jnp.minimum(wid, rem)
        cnt = base + (wid < rem).astype(jnp.int32)

        pltpu.sync_copy(spk_h, spk)
        pltpu.sync_copy(pk_h, pk)

        @pl.when(sid == 0)
        def _():
            pltpu.sync_copy(zeros_h, acc)
        plsc.subcore_barrier()

        ins = (src_h, dst_h, d_h, sw_h)
        bufs0 = (srcb0, dstb0, db0, swb0)
        bufs1 = (srcb1, dstb1, db1, swb1)

        def fire_in(c, bufs, sem):
            row = (start + c) * NBLK
            for h, b in zip(ins, bufs):
                pltpu.async_copy(h.at[pl.ds(row, NBLK)], b, sem)

        def wait_in(bufs, sem):
            for h, b in zip(ins, bufs):
                pltpu.make_async_copy(h.at[pl.ds(0, NBLK)], b, sem).wait()

        def fire_sc(valb, scb, sem):
            for j in range(NBLK):
                pltpu.async_copy(valb.at[pl.ds(j * BLK, BLK)],
                                 acc.at[scb.at[j]], sem, add=True)

        def wait_sc(valb, scb, sem):
            for j in range(NBLK):
                pltpu.make_async_copy(valb.at[pl.ds(j * BLK, BLK)],
                                      acc.at[scb.at[j]], sem).wait()

        half = 0.5 * BOHR / 65536.0

        def compute(srcb, dstb, db, swb, valb, scb):
            @plsc.parallel_loop(0, CHUNK // LANES, 1, unroll=UNROLL)
            def _(g):
                j = lax.shift_right_logical(g, 3)
                sl = pl.ds((g & 7) * LANES, LANES)
                s = srcb[j, sl]
                t = dstb[j, sl]
                zs_p = plsc.load_gather(spk, [lax.shift_right_logical(s, 2)])
                zt_p = plsc.load_gather(spk, [lax.shift_right_logical(t, 2)])
                zs = lax.shift_right_logical(zs_p, (s & 3) * 8) & 255
                zt = lax.shift_right_logical(zt_p, (t & 3) * 8) & 255
                b3 = (zs + ZMAX * zt) * 3
                d = db[j, sl]
                nd = d * (-1.0 / 8192.0)
                v0 = plsc.load_gather(pk, [b3])
                v1 = plsc.load_gather(pk, [b3 + 1])
                v2 = plsc.load_gather(pk, [b3 + 2])
                phi = (lax.shift_right_logical(v0, 16).astype(jnp.float32)
                       * jnp.exp((v0 & 0xFFFF).astype(jnp.float32) * nd)
                       + lax.shift_right_logical(v1, 16).astype(jnp.float32)
                       * jnp.exp((v1 & 0xFFFF).astype(jnp.float32) * nd)
                       + lax.shift_right_logical(v2, 16).astype(jnp.float32)
                       * jnp.exp((v2 & 0xFFFF).astype(jnp.float32) * nd))
                zz = (zs * zt).astype(jnp.float32)
                scb[j, sl] = s
                valb[pl.ds(g * LANES, LANES)] = zz * phi * swb[j, sl] * half / d

        @pl.when(cnt > 0)
        def _():
            fire_in(0, bufs0, sem0)

        npairs = (cnt + 1) // 2

        def pair(p, carry):
            c1 = 2 * p + 1
            c2 = 2 * p + 2

            @pl.when(c1 < cnt)
            def _():
                fire_in(c1, bufs1, sem1)

            wait_in(bufs0, sem0)

            @pl.when(p > 0)
            def _():
                wait_sc(valb0, scb0, scsem0)
            compute(srcb0, dstb0, db0, swb0, valb0, scb0)
            fire_sc(valb0, scb0, scsem0)

            @pl.when(c2 < cnt)
            def _():
                fire_in(c2, bufs0, sem0)

            @pl.when(c1 < cnt)
            def _():
                wait_in(bufs1, sem1)

                @pl.when(p > 0)
                def _():
                    wait_sc(valb1, scb1, scsem1)
                compute(srcb1, dstb1, db1, swb1, valb1, scb1)
                fire_sc(valb1, scb1, scsem1)
            return carry

        lax.fori_loop(0, npairs, pair, 0)

        @pl.when(cnt >= 1)
        def _():
            wait_sc(valb0, scb0, scsem0)

        @pl.when(cnt >= 2)
        def _():
            wait_sc(valb1, scb1, scsem1)

        plsc.subcore_barrier()

        @pl.when(sid == 0)
        def _():
            pltpu.sync_copy(acc, out_h.at[cid])

    f32 = jnp.float32
    i32 = jnp.int32
    mesh = plsc.VectorSubcoreMesh(core_axis_name="c", subcore_axis_name="s",
                                  num_cores=NC, num_subcores=NS)
    ebuf_i = pltpu.VMEM((NBLK, BLK), i32)
    ebuf_f = pltpu.VMEM((NBLK, BLK), f32)
    vbuf_f = pltpu.VMEM((CHUNK,), f32)
    return pl.kernel(
        body,
        out_type=jax.ShapeDtypeStruct((NC, n_nodes), f32),
        mesh=mesh,
        compiler_params=pltpu.CompilerParams(needs_layout_passes=False),
        scratch_types=[
            pltpu.VMEM((spk_n,), i32),
            pltpu.VMEM((tab_n,), i32),
            ebuf_i, ebuf_i, ebuf_i, ebuf_i, ebuf_f, ebuf_f, ebuf_f, ebuf_f,
            vbuf_f, vbuf_f, ebuf_i, ebuf_i,
            pltpu.VMEM_SHARED((n_nodes,), f32),
            pltpu.SemaphoreType.DMA,
            pltpu.SemaphoreType.DMA,
            pltpu.SemaphoreType.DMA,
            pltpu.SemaphoreType.DMA,
        ],
    )


def kernel(species, edge_src, edge_dst, distances, switch, CS, ALPHAS):
    n_nodes = species.shape[0]
    n_edges = edge_src.shape[0]
    i32 = jnp.int32

    sp = _pad1d(species.astype(i32), 4).reshape(-1, 4)
    packed = (sp[:, 0] | (sp[:, 1] << 8) | (sp[:, 2] << 16) | (sp[:, 3] << 24))
    packed = _pad1d(packed, 16)

    cs_q = jnp.minimum(jnp.round(CS.reshape(-1) * 65536.0), 65535.0)
    al_q = jnp.minimum(jnp.round(ALPHAS.reshape(-1) * 8192.0), 65535.0)
    pk = (cs_q.astype(i32) << 16) | al_q.astype(i32)
    pk = _pad1d(pk, 32)

    src2 = edge_src.astype(i32).reshape(-1, BLK)
    dst2 = edge_dst.astype(i32).reshape(-1, BLK)
    d2 = distances.reshape(-1, BLK)
    sw2 = switch.reshape(-1, BLK)
    zeros = jnp.zeros((n_nodes,), jnp.float32)

    call = _make_sc_call(n_nodes, n_edges, packed.shape[0], pk.shape[0])
    out = call(packed, pk, src2, dst2, d2, sw2, zeros)
    return out[0] + out[1]


# speedup vs baseline: 1.8560x; 1.0141x over previous
import jax
import jax.numpy as jnp
from jax import lax
from jax.experimental import pallas as pl
from jax.experimental.pallas import tpu as pltpu
from jax.experimental.pallas import tpu_sc as plsc

BOHR = 0.52917721067121
ZMAX = 92
NC = 2
NS = 16
NW = NC * NS
LANES = 16
BLK = 128
NBLK = 16
CHUNK = NBLK * BLK
UNROLL = 4


def _pad1d(x, mult):
    n = x.shape[0]
    p = (-n) % mult
    if p:
        x = jnp.concatenate([x, jnp.zeros((p,), x.dtype)])
    return x


def _make_sc_call(n_nodes, n_edges, spk_n, tab_n):
    nch = n_edges // CHUNK
    base, rem = nch // NW, nch % NW

    def body(spk_h, pk_h, src_h, dst_h, d_h, sw_h, zeros_h,
             out_h,
             spk, pk,
             srcb0, srcb1, dstb0, dstb1, db0, db1, swb0, swb1,
             valb0, valb1, scb0, scb1, b3b, zzb,
             acc, sem0, sem1, scsem0, scsem1):
        cid = lax.axis_index("c")
        sid = lax.axis_index("s")
        wid = sid * NC + cid
        start = wid * base + jnp.minimum(wid, rem)
        cnt = base + (wid < rem).astype(jnp.int32)

        pltpu.sync_copy(spk_h, spk)
        pltpu.sync_copy(pk_h, pk)

        @pl.when(sid == 0)
        def _():
            pltpu.sync_copy(zeros_h, acc)
        plsc.subcore_barrier()

        ins = (src_h, dst_h, d_h, sw_h)
        bufs0 = (srcb0, dstb0, db0, swb0)
        bufs1 = (srcb1, dstb1, db1, swb1)

        def fire_in(c, bufs, sem):
            row = (start + c) * NBLK
            for h, b in zip(ins, bufs):
                pltpu.async_copy(h.at[pl.ds(row, NBLK)], b, sem)

        def wait_in(bufs, sem):
            for h, b in zip(ins, bufs):
                pltpu.make_async_copy(h.at[pl.ds(0, NBLK)], b, sem).wait()

        def fire_sc(valb, scb, sem):
            for j in range(NBLK):
                pltpu.async_copy(valb.at[pl.ds(j * BLK, BLK)],
                                 acc.at[scb.at[j]], sem, add=True)

        def wait_sc(valb, scb, sem):
            for j in range(NBLK):
                pltpu.make_async_copy(valb.at[pl.ds(j * BLK, BLK)],
                                      acc.at[scb.at[j]], sem).wait()

        half = 0.5 * BOHR / 65536.0

        def compute(srcb, dstb, db, swb, valb, scb, b3b, zzb):
            @plsc.parallel_loop(0, CHUNK // LANES, 1, unroll=UNROLL)
            def _(g):
                j = lax.shift_right_logical(g, 3)
                sl = pl.ds((g & 7) * LANES, LANES)
                gl = pl.ds(g * LANES, LANES)
                s = srcb[j, sl]
                t = dstb[j, sl]
                zs_p = plsc.load_gather(spk, [lax.shift_right_logical(s, 2)])
                zt_p = plsc.load_gather(spk, [lax.shift_right_logical(t, 2)])
                zs = lax.shift_right_logical(zs_p, (s & 3) * 8) & 255
                zt = lax.shift_right_logical(zt_p, (t & 3) * 8) & 255
                scb[j, sl] = s
                b3b[gl] = (zs + ZMAX * zt) * 3
                zzb[gl] = (zs * zt).astype(jnp.float32)

            @plsc.parallel_loop(0, CHUNK // LANES, 1, unroll=UNROLL)
            def _(g):
                j = lax.shift_right_logical(g, 3)
                sl = pl.ds((g & 7) * LANES, LANES)
                gl = pl.ds(g * LANES, LANES)
                b3 = b3b[gl]
                d = db[j, sl]
                nd = d * (-1.0 / 8192.0)
                v0 = plsc.load_gather(pk, [b3])
                v1 = plsc.load_gather(pk, [b3 + 1])
                v2 = plsc.load_gather(pk, [b3 + 2])
                phi = (lax.shift_right_logical(v0, 16).astype(jnp.float32)
                       * jnp.exp((v0 & 0xFFFF).astype(jnp.float32) * nd)
                       + lax.shift_right_logical(v1, 16).astype(jnp.float32)
                       * jnp.exp((v1 & 0xFFFF).astype(jnp.float32) * nd)
                       + lax.shift_right_logical(v2, 16).astype(jnp.float32)
                       * jnp.exp((v2 & 0xFFFF).astype(jnp.float32) * nd))
                valb[gl] = zzb[gl] * phi * swb[j, sl] * half / d

        @pl.when(cnt > 0)
        def _():
            fire_in(0, bufs0, sem0)

        npairs = (cnt + 1) // 2

        def pair(p, carry):
            c1 = 2 * p + 1
            c2 = 2 * p + 2

            @pl.when(c1 < cnt)
            def _():
                fire_in(c1, bufs1, sem1)

            wait_in(bufs0, sem0)

            @pl.when(p > 0)
            def _():
                wait_sc(valb0, scb0, scsem0)
            compute(srcb0, dstb0, db0, swb0, valb0, scb0, b3b, zzb)
            fire_sc(valb0, scb0, scsem0)

            @pl.when(c2 < cnt)
            def _():
                fire_in(c2, bufs0, sem0)

            @pl.when(c1 < cnt)
            def _():
                wait_in(bufs1, sem1)

                @pl.when(p > 0)
                def _():
                    wait_sc(valb1, scb1, scsem1)
                compute(srcb1, dstb1, db1, swb1, valb1, scb1, b3b, zzb)
                fire_sc(valb1, scb1, scsem1)
            return carry

        lax.fori_loop(0, npairs, pair, 0)

        @pl.when(cnt >= 1)
        def _():
            wait_sc(valb0, scb0, scsem0)

        @pl.when(cnt >= 2)
        def _():
            wait_sc(valb1, scb1, scsem1)

        plsc.subcore_barrier()

        @pl.when(sid == 0)
        def _():
            pltpu.sync_copy(acc, out_h.at[cid])

    f32 = jnp.float32
    i32 = jnp.int32
    mesh = plsc.VectorSubcoreMesh(core_axis_name="c", subcore_axis_name="s",
                                  num_cores=NC, num_subcores=NS)
    ebuf_i = pltpu.VMEM((NBLK, BLK), i32)
    ebuf_f = pltpu.VMEM((NBLK, BLK), f32)
    vbuf_f = pltpu.VMEM((CHUNK,), f32)
    return pl.kernel(
        body,
        out_type=jax.ShapeDtypeStruct((NC, n_nodes), f32),
        mesh=mesh,
        compiler_params=pltpu.CompilerParams(needs_layout_passes=False),
        scratch_types=[
            pltpu.VMEM((spk_n,), i32),
            pltpu.VMEM((tab_n,), i32),
            ebuf_i, ebuf_i, ebuf_i, ebuf_i, ebuf_f, ebuf_f, ebuf_f, ebuf_f,
            vbuf_f, vbuf_f, ebuf_i, ebuf_i,
            pltpu.VMEM((CHUNK,), i32), vbuf_f,
            pltpu.VMEM_SHARED((n_nodes,), f32),
            pltpu.SemaphoreType.DMA,
            pltpu.SemaphoreType.DMA,
            pltpu.SemaphoreType.DMA,
            pltpu.SemaphoreType.DMA,
        ],
    )


def kernel(species, edge_src, edge_dst, distances, switch, CS, ALPHAS):
    n_nodes = species.shape[0]
    n_edges = edge_src.shape[0]
    i32 = jnp.int32

    sp = _pad1d(species.astype(i32), 4).reshape(-1, 4)
    packed = (sp[:, 0] | (sp[:, 1] << 8) | (sp[:, 2] << 16) | (sp[:, 3] << 24))
    packed = _pad1d(packed, 16)

    cs_q = jnp.minimum(jnp.round(CS.reshape(-1) * 65536.0), 65535.0)
    al_q = jnp.minimum(jnp.round(ALPHAS.reshape(-1) * 8192.0), 65535.0)
    pk = (cs_q.astype(i32) << 16) | al_q.astype(i32)
    pk = _pad1d(pk, 32)

    src2 = edge_src.astype(i32).reshape(-1, BLK)
    dst2 = edge_dst.astype(i32).reshape(-1, BLK)
    d2 = distances.reshape(-1, BLK)
    sw2 = switch.reshape(-1, BLK)
    zeros = jnp.zeros((n_nodes,), jnp.float32)

    call = _make_sc_call(n_nodes, n_edges, packed.shape[0], pk.shape[0])
    out = call(packed, pk, src2, dst2, d2, sw2, zeros)
    return out[0] + out[1]


# speedup vs baseline: 2.0611x; 1.1105x over previous
import jax
import jax.numpy as jnp
from jax import lax
from jax.experimental import pallas as pl
from jax.experimental.pallas import tpu as pltpu
from jax.experimental.pallas import tpu_sc as plsc

BOHR = 0.52917721067121
ZMAX = 92
NC = 2
NS = 16
NW = NC * NS
LANES = 16
BLK = 128
NBLK = 16
CHUNK = NBLK * BLK
UNROLL = 4


def _pad1d(x, mult):
    n = x.shape[0]
    p = (-n) % mult
    if p:
        x = jnp.concatenate([x, jnp.zeros((p,), x.dtype)])
    return x


def _make_sc_call(n_nodes, n_edges, spk_n, tab_n):
    nch = n_edges // CHUNK
    base, rem = nch // NW, nch % NW

    def body(spk_h, csf_h, alf_h, src_h, dst_h, d_h, sw_h, zeros_h,
             out_h,
             spk, csf, alf,
             srcb0, srcb1, dstb0, dstb1, db0, db1, swb0, swb1,
             valb0, valb1, scb0, scb1,
             acc, sem0, sem1, scsem0, scsem1):
        cid = lax.axis_index("c")
        sid = lax.axis_index("s")
        wid = sid * NC + cid
        start = wid * base + jnp.minimum(wid, rem)
        cnt = base + (wid < rem).astype(jnp.int32)

        pltpu.sync_copy(spk_h, spk)
        pltpu.sync_copy(csf_h, csf)
        pltpu.sync_copy(alf_h, alf)

        @pl.when(sid == 0)
        def _():
            pltpu.sync_copy(zeros_h, acc)
        plsc.subcore_barrier()

        ins = (src_h, dst_h, d_h, sw_h)
        bufs0 = (srcb0, dstb0, db0, swb0)
        bufs1 = (srcb1, dstb1, db1, swb1)

        def fire_in(c, bufs, sem):
            row = (start + c) * NBLK
            for h, b in zip(ins, bufs):
                pltpu.async_copy(h.at[pl.ds(row, NBLK)], b, sem)

        def wait_in(bufs, sem):
            for h, b in zip(ins, bufs):
                pltpu.make_async_copy(h.at[pl.ds(0, NBLK)], b, sem).wait()

        def fire_sc(valb, scb, sem):
            for j in range(NBLK):
                pltpu.async_copy(valb.at[pl.ds(j * BLK, BLK)],
                                 acc.at[scb.at[j]], sem, add=True)

        def wait_sc(valb, scb, sem):
            for j in range(NBLK):
                pltpu.make_async_copy(valb.at[pl.ds(j * BLK, BLK)],
                                      acc.at[scb.at[j]], sem).wait()

        half = 0.5 * BOHR

        def compute(srcb, dstb, db, swb, valb, scb):
            @plsc.parallel_loop(0, CHUNK // LANES, 1, unroll=UNROLL)
            def _(g):
                j = lax.shift_right_logical(g, 3)
                sl = pl.ds((g & 7) * LANES, LANES)
                s = srcb[j, sl]
                t = dstb[j, sl]
                zs_p = plsc.load_gather(spk, [lax.shift_right_logical(s, 2)])
                zt_p = plsc.load_gather(spk, [lax.shift_right_logical(t, 2)])
                zs = lax.shift_right_logical(zs_p, (s & 3) * 8) & 255
                zt = lax.shift_right_logical(zt_p, (t & 3) * 8) & 255
                b3 = (zs + ZMAX * zt) * 3
                d = db[j, sl]
                nd = -d
                phi = (plsc.load_gather(csf, [b3])
                       * jnp.exp(plsc.load_gather(alf, [b3]) * nd)
                       + plsc.load_gather(csf, [b3 + 1])
                       * jnp.exp(plsc.load_gather(alf, [b3 + 1]) * nd)
                       + plsc.load_gather(csf, [b3 + 2])
                       * jnp.exp(plsc.load_gather(alf, [b3 + 2]) * nd))
                zz = (zs * zt).astype(jnp.float32)
                scb[j, sl] = s
                valb[pl.ds(g * LANES, LANES)] = zz * phi * swb[j, sl] * half / d

        @pl.when(cnt > 0)
        def _():
            fire_in(0, bufs0, sem0)

        npairs = (cnt + 1) // 2

        def pair(p, carry):
            c1 = 2 * p + 1
            c2 = 2 * p + 2

            @pl.when(c1 < cnt)
            def _():
                fire_in(c1, bufs1, sem1)

            wait_in(bufs0, sem0)

            @pl.when(p > 0)
            def _():
                wait_sc(valb0, scb0, scsem0)
            compute(srcb0, dstb0, db0, swb0, valb0, scb0)
            fire_sc(valb0, scb0, scsem0)

            @pl.when(c2 < cnt)
            def _():
                fire_in(c2, bufs0, sem0)

            @pl.when(c1 < cnt)
            def _():
                wait_in(bufs1, sem1)

                @pl.when(p > 0)
                def _():
                    wait_sc(valb1, scb1, scsem1)
                compute(srcb1, dstb1, db1, swb1, valb1, scb1)
                fire_sc(valb1, scb1, scsem1)
            return carry

        lax.fori_loop(0, npairs, pair, 0)

        @pl.when(cnt >= 1)
        def _():
            wait_sc(valb0, scb0, scsem0)

        @pl.when(cnt >= 2)
        def _():
            wait_sc(valb1, scb1, scsem1)

        plsc.subcore_barrier()

        @pl.when(sid == 0)
        def _():
            pltpu.sync_copy(acc, out_h.at[cid])

    f32 = jnp.float32
    i32 = jnp.int32
    mesh = plsc.VectorSubcoreMesh(core_axis_name="c", subcore_axis_name="s",
                                  num_cores=NC, num_subcores=NS)
    ebuf_i = pltpu.VMEM((NBLK, BLK), i32)
    ebuf_f = pltpu.VMEM((NBLK, BLK), f32)
    vbuf_f = pltpu.VMEM((CHUNK,), f32)
    return pl.kernel(
        body,
        out_type=jax.ShapeDtypeStruct((NC, n_nodes), f32),
        mesh=mesh,
        compiler_params=pltpu.CompilerParams(needs_layout_passes=False),
        scratch_types=[
            pltpu.VMEM((spk_n,), i32),
            pltpu.VMEM((tab_n,), f32),
            pltpu.VMEM((tab_n,), f32),
            ebuf_i, ebuf_i, ebuf_i, ebuf_i, ebuf_f, ebuf_f, ebuf_f, ebuf_f,
            vbuf_f, vbuf_f, ebuf_i, ebuf_i,
            pltpu.VMEM_SHARED((n_nodes,), f32),
            pltpu.SemaphoreType.DMA,
            pltpu.SemaphoreType.DMA,
            pltpu.SemaphoreType.DMA,
            pltpu.SemaphoreType.DMA,
        ],
    )


def kernel(species, edge_src, edge_dst, distances, switch, CS, ALPHAS):
    n_nodes = species.shape[0]
    n_edges = edge_src.shape[0]
    i32 = jnp.int32

    sp = _pad1d(species.astype(i32), 4).reshape(-1, 4)
    packed = (sp[:, 0] | (sp[:, 1] << 8) | (sp[:, 2] << 16) | (sp[:, 3] << 24))
    packed = _pad1d(packed, 16)

    csf = _pad1d(CS.reshape(-1), 32)
    alf = _pad1d(ALPHAS.reshape(-1), 32)

    src2 = edge_src.astype(i32).reshape(-1, BLK)
    dst2 = edge_dst.astype(i32).reshape(-1, BLK)
    d2 = distances.reshape(-1, BLK)
    sw2 = switch.reshape(-1, BLK)
    zeros = jnp.zeros((n_nodes,), jnp.float32)

    call = _make_sc_call(n_nodes, n_edges, packed.shape[0], csf.shape[0])
    out = call(packed, csf, alf, src2, dst2, d2, sw2, zeros)
    return out[0] + out[1]
